# baseline (speedup 1.0000x reference)
import functools

import jax
import jax.numpy as jnp
from jax import lax
from jax.experimental import pallas as pl
from jax.experimental.pallas import tpu as pltpu
from jax.experimental.pallas import tpu_sc as plsc

VOCAB = 1_000_000
EMBED = 32
BATCH = 4096
LABELS = 50

NC = 2
NS = 16
NW = NC * NS

ROWS_PER_W = BATCH // NW
IDXW = 100
IDX_ROWS = BATCH * LABELS // IDXW
IDX_ROWS_PER_W = IDX_ROWS // NW
K = 16
CHUNKS = IDX_ROWS_PER_W // K
BR_PER_CHUNK = K * IDXW // LABELS


def _sc_body(emb_hbm, idx_hbm, out_hbm, idx_v, rows_v, out_v, sem):
    wid = lax.axis_index("s") * NC + lax.axis_index("c")
    for c in range(CHUNKS):
        row0 = wid * IDX_ROWS_PER_W + c * K
        pltpu.sync_copy(idx_hbm.at[pl.ds(row0, K)], idx_v)
        copies = [
            pltpu.async_copy(
                emb_hbm.at[idx_v.at[j]],
                rows_v.at[pl.ds(j * IDXW, IDXW)],
                sem,
            )
            for j in range(K)
        ]
        for cp in copies:
            cp.wait()

        def body(r, carry):
            base = r * LABELS
            a0 = rows_v[base, 0:16]
            a1 = rows_v[base, 16:32]
            for l in range(1, LABELS):
                a0 = a0 + rows_v[base + l, 0:16]
                a1 = a1 + rows_v[base + l, 16:32]
            out_v[r, 0:16] = a0
            out_v[r, 16:32] = a1
            return carry

        lax.fori_loop(0, BR_PER_CHUNK, body, 0)
        out0 = wid * ROWS_PER_W + c * BR_PER_CHUNK
        pltpu.sync_copy(out_v, out_hbm.at[pl.ds(out0, BR_PER_CHUNK)])


@jax.jit
def _run(inputs, emb):
    idx = inputs.reshape(IDX_ROWS, IDXW).astype(jnp.int32)
    mesh = plsc.VectorSubcoreMesh(core_axis_name="c", subcore_axis_name="s")
    f = functools.partial(
        pl.kernel,
        mesh=mesh,
        compiler_params=pltpu.CompilerParams(use_tc_tiling_on_sc=False),
        out_type=jax.ShapeDtypeStruct((BATCH, EMBED), jnp.float32),
        scratch_types=[
            pltpu.VMEM((K, IDXW), jnp.int32),
            pltpu.VMEM((K * IDXW, EMBED), jnp.float32),
            pltpu.VMEM((BR_PER_CHUNK, EMBED), jnp.float32),
            pltpu.SemaphoreType.DMA,
        ],
    )(_sc_body)
    return f(emb, idx)


def kernel(inputs, emb):
    return _run(inputs, emb)
